# baseline (speedup 1.0000x reference)
import functools

import jax
import jax.numpy as jnp
from jax import lax
from jax.experimental import pallas as pl
from jax.experimental.pallas import tpu as pltpu
from jax.experimental.pallas import tpu_sc as plsc

N = 10000
E = 320000
H = 128

_NC = 2
_NS = 16
_NW = _NC * _NS
_EW = E // _NW

_RPT = 624
_RTAIL = N - _NS * _RPT


def _leaky(x):
    return jnp.where(x >= 0, x, 0.1 * x)


def _dot(a, b):
    return jnp.dot(a, b, preferred_element_type=jnp.float32)


def _node_pre_body(x, w1, b1, w2, b2, wa, o_ref):
    h = _leaky(_dot(x[...], w1[...]) + b1[...])
    nf = _leaky(_dot(h, w2[...]) + b2[...])
    a = _dot(nf, wa[...])
    o_ref[...] = jnp.concatenate([a, jnp.zeros_like(a)], axis=1)


def _node_pre(node_feat, w1, b1, w2, b2, wa):
    bn = 1000
    return pl.pallas_call(
        _node_pre_body,
        grid=(N // bn,),
        in_specs=[
            pl.BlockSpec((bn, 128), lambda i: (i, 0)),
            pl.BlockSpec((128, 64), lambda i: (0, 0)),
            pl.BlockSpec((1, 64), lambda i: (0, 0)),
            pl.BlockSpec((64, 128), lambda i: (0, 0)),
            pl.BlockSpec((1, 128), lambda i: (0, 0)),
            pl.BlockSpec((128, 64), lambda i: (0, 0)),
        ],
        out_specs=pl.BlockSpec((bn, 128), lambda i: (i, 0)),
        out_shape=jax.ShapeDtypeStruct((N, 128), jnp.float32),
    )(node_feat, w1, b1, w2, b2, wa)


def _edge_mlp_body(xt, w1e, b1e, w2e, b2e, we1, be1, we2, be2,
                   wb1, bb1, wb2, bb2,
                   ef_ref, e1_ref, e2_ref):
    h = _leaky(jax.lax.dot_general(xt[...], w1e[...],
                                   (((0,), (0,)), ((), ())),
                                   preferred_element_type=jnp.float32)
               + b1e[...])
    ef = _leaky(_dot(h, w2e[...]) + b2e[...])
    ef_ref[...] = ef
    e1_ref[...] = _dot(ef, wb1[...]) + bb1[...]
    h2 = _leaky(_dot(ef, we1[...]) + be1[...])
    ef2 = _leaky(_dot(h2, we2[...]) + be2[...])
    e2_ref[...] = _dot(ef2, wb2[...]) + bb2[...]


def _edge_mlp(edge_feat_t, w1e, b1e, w2e, b2e, we1, be1, we2, be2,
              wb1, bb1, wb2, bb2):
    be = 2560
    full = lambda r, c: pl.BlockSpec((r, c), lambda i: (0, 0))
    return pl.pallas_call(
        _edge_mlp_body,
        grid=(E // be,),
        in_specs=[
            pl.BlockSpec((16, be), lambda i: (0, i)),
            full(16, 64), full(1, 64), full(64, 128), full(1, 128),
            full(128, 64), full(1, 64), full(64, 128), full(1, 128),
            full(128, 64), full(1, 64), full(128, 64), full(1, 64),
        ],
        out_specs=[
            pl.BlockSpec((be, 128), lambda i: (i, 0)),
            pl.BlockSpec((be, 64), lambda i: (i, 0)),
            pl.BlockSpec((be, 64), lambda i: (i, 0)),
        ],
        out_shape=[
            jax.ShapeDtypeStruct((E, 128), jnp.float32),
            jax.ShapeDtypeStruct((E, 64), jnp.float32),
            jax.ShapeDtypeStruct((E, 64), jnp.float32),
        ],
    )(edge_feat_t, w1e, b1e, w2e, b2e, we1, be1, we2, be2,
      wb1, bb1, wb2, bb2)


def _msg2_body(g2, e2, w22, b22, m_ref):
    hm = _leaky(g2[...][:, :64] + e2[...])
    m_ref[...] = _leaky(_dot(hm, w22[...]) + b22[...])


def _msg2(g2, e2, w22, b22):
    be = 2000
    return pl.pallas_call(
        _msg2_body,
        grid=(E // be,),
        in_specs=[
            pl.BlockSpec((be, 128), lambda i: (i, 0)),
            pl.BlockSpec((be, 64), lambda i: (i, 0)),
            pl.BlockSpec((64, 128), lambda i: (0, 0)),
            pl.BlockSpec((1, 128), lambda i: (0, 0)),
        ],
        out_specs=pl.BlockSpec((be, 128), lambda i: (i, 0)),
        out_shape=jax.ShapeDtypeStruct((E, 128), jnp.float32),
    )(g2, e2, w22, b22)


def _node_pre2_body(p, wa, o_ref):
    a = _dot(p[0] + p[1], wa[...])
    o_ref[...] = jnp.concatenate([a, jnp.zeros_like(a)], axis=1)


def _node_pre2(p, wa):
    bn = 1000
    return pl.pallas_call(
        _node_pre2_body,
        grid=(N // bn,),
        in_specs=[
            pl.BlockSpec((2, bn, 128), lambda i: (0, i, 0)),
            pl.BlockSpec((128, 64), lambda i: (0, 0)),
        ],
        out_specs=pl.BlockSpec((bn, 128), lambda i: (i, 0)),
        out_shape=jax.ShapeDtypeStruct((N, 128), jnp.float32),
    )(p, wa)


def _graph_body(xq, w, b, o_ref):
    k = pl.program_id(0)

    @pl.when(k == 0)
    def _():
        o_ref[...] = jnp.zeros_like(o_ref)

    x = xq[0:1, :] + xq[1:2, :]
    o_ref[...] += _dot(x, w[...])

    @pl.when(k == pl.num_programs(0) - 1)
    def _():
        o_ref[...] = _leaky(o_ref[...] + b[...])


def _graph_linear(q2, graph_W, graph_b):
    bk = 12800
    k_total = N * H
    return pl.pallas_call(
        _graph_body,
        grid=(k_total // bk,),
        in_specs=[
            pl.BlockSpec((2, bk), lambda k: (0, k)),
            pl.BlockSpec((bk, 128), lambda k: (k, 0)),
            pl.BlockSpec((1, 128), lambda k: (0, 0)),
        ],
        out_specs=pl.BlockSpec((1, 128), lambda k: (0, 0)),
        out_shape=jax.ShapeDtypeStruct((1, 128), jnp.float32),
    )(q2, graph_W, graph_b)


def _sc_gather(table, dst):
    mesh = plsc.VectorSubcoreMesh(core_axis_name="c", subcore_axis_name="s")
    chunk = 400
    nch = _EW // chunk

    @functools.partial(
        pl.kernel,
        out_type=jax.ShapeDtypeStruct((E, H), jnp.float32),
        mesh=mesh,
        scratch_types=[
            pltpu.VMEM((chunk,), jnp.int32),
            pltpu.VMEM((chunk,), jnp.int32),
            pltpu.VMEM((chunk, H), jnp.float32),
            pltpu.VMEM((chunk, H), jnp.float32),
            pltpu.SemaphoreType.DMA,
            pltpu.SemaphoreType.DMA,
        ],
    )
    def k(table_hbm, ei_hbm, out_hbm, idx0, idx1, rows0, rows1, sem0, sem1):
        wid = lax.axis_index("s") * _NC + lax.axis_index("c")
        base = wid * _EW
        rows = (rows0, rows1)
        idxs = (idx0, idx1)
        sems = (sem0, sem1)

        def body(g, carry):
            gbase = base + g * (chunk * 5)
            descs = [None, None]
            pltpu.sync_copy(ei_hbm.at[pl.ds(gbase, chunk)], idx0)
            descs[0] = pltpu.async_copy(table_hbm.at[idx0], rows0, sem0)
            for j in range(5):
                b, nb = j % 2, (j + 1) % 2
                if j + 1 < 5:
                    off = gbase + (j + 1) * chunk
                    pltpu.sync_copy(ei_hbm.at[pl.ds(off, chunk)], idxs[nb])
                    descs[nb] = pltpu.async_copy(table_hbm.at[idxs[nb]],
                                                 rows[nb], sems[nb])
                descs[b].wait()
                pltpu.sync_copy(rows[b],
                                out_hbm.at[pl.ds(gbase + j * chunk, chunk)])
            return carry

        lax.fori_loop(0, nch // 5, body, 0)

    return k(table, dst)


def _sc_scatter(m, dst):
    mesh = plsc.VectorSubcoreMesh(core_axis_name="c", subcore_axis_name="s")
    chunk, unroll = 80, 5
    outer = _EW // (chunk * unroll)
    zrows = jnp.zeros((16, H), jnp.float32)

    @functools.partial(
        pl.kernel,
        out_type=jax.ShapeDtypeStruct((_NC, N, H), jnp.float32),
        mesh=mesh,
        scratch_types=[
            pltpu.VMEM((chunk,), jnp.int32),
            pltpu.VMEM((chunk,), jnp.int32),
            pltpu.VMEM((chunk, H), jnp.float32),
            pltpu.VMEM((chunk, H), jnp.float32),
            pltpu.VMEM_SHARED((N, H), jnp.float32),
            pltpu.SemaphoreType.DMA,
            pltpu.SemaphoreType.DMA,
        ],
    )
    def k(m_hbm, ei_hbm, z_hbm, out_hbm, idx0, idx1, rows0, rows1, acc_sh,
          sem0, sem1):
        c = lax.axis_index("c")
        s = lax.axis_index("s")
        wid = s * _NC + c
        def zbody(j, carry):
            pltpu.sync_copy(z_hbm, acc_sh.at[pl.ds(s * _RPT + j * 16, 16)])
            return carry

        lax.fori_loop(0, _RPT // 16, zbody, 0)

        @pl.when(s == _NS - 1)
        def _():
            pltpu.sync_copy(z_hbm, acc_sh.at[pl.ds(_NS * _RPT, _RTAIL)])

        plsc.subcore_barrier()

        base = wid * _EW
        rows = (rows0, rows1)
        idxs = (idx0, idx1)
        sems = (sem0, sem1)

        def body(g, carry):
            gbase = base + g * (chunk * unroll)
            descs = [None, None]
            pltpu.sync_copy(ei_hbm.at[pl.ds(gbase, chunk)], idx0)
            descs[0] = pltpu.async_copy(m_hbm.at[pl.ds(gbase, chunk)],
                                        rows0, sem0)
            for j in range(unroll):
                b, nb = j % 2, (j + 1) % 2
                if j + 1 < unroll:
                    off = gbase + (j + 1) * chunk
                    pltpu.sync_copy(ei_hbm.at[pl.ds(off, chunk)], idxs[nb])
                    descs[nb] = pltpu.async_copy(m_hbm.at[pl.ds(off, chunk)],
                                                 rows[nb], sems[nb])
                descs[b].wait()
                pltpu.sync_copy(rows[b], acc_sh.at[idxs[b]], add=True)
            return carry

        lax.fori_loop(0, outer, body, 0)
        plsc.subcore_barrier()
        pltpu.sync_copy(acc_sh.at[pl.ds(s * _RPT, _RPT)],
                        out_hbm.at[c, pl.ds(s * _RPT, _RPT)])

        @pl.when(s == _NS - 1)
        def _():
            pltpu.sync_copy(acc_sh.at[pl.ds(_NS * _RPT, _RTAIL)],
                            out_hbm.at[c, pl.ds(_NS * _RPT, _RTAIL)])

    return k(m, dst, zrows)


def kernel(node_feat, edge_feat, edge_index, mlp_node, mlp_edge,
           gnn1_mlp, gnn2_mlp, gnn2_mlp_edge, graph_W, graph_b):
    w1n, b1n, w2n, b2n = mlp_node
    w1e, b1e, w2e, b2e = mlp_edge
    w11, b11, w21, b21 = gnn1_mlp
    w12, b12, w22, b22 = gnn2_mlp
    we1, be1, we2, be2 = gnn2_mlp_edge

    r = lambda v: v.reshape(1, -1)
    wa1, wb1 = w11[:H], w11[H:]
    wa2, wb2 = w12[:H], w12[H:]

    dst = edge_index[1].astype(jnp.int32)

    a1 = _node_pre(node_feat, w1n, r(b1n), w2n, r(b2n), wa1)
    g1 = _sc_gather(a1, dst)

    ef, e1, e2 = _edge_mlp(edge_feat.T, w1e, r(b1e), w2e, r(b2e),
                           we1, r(be1), we2, r(be2),
                           wb1, r(b11), wb2, r(b12))
    m1 = _msg2(g1, e1, w21, r(b21))
    p = _sc_scatter(m1, dst)

    a2 = _node_pre2(p, wa2)
    g2 = _sc_gather(a2, dst)
    m2 = _msg2(g2, e2, w22, r(b22))
    q = _sc_scatter(m2, dst)

    g = _graph_linear(q.reshape(_NC, N * H), graph_W, r(graph_b))
    return (g.reshape(H), ef)


# speedup vs baseline: 1.0526x; 1.0526x over previous
import functools

import jax
import jax.numpy as jnp
from jax import lax
from jax.experimental import pallas as pl
from jax.experimental.pallas import tpu as pltpu
from jax.experimental.pallas import tpu_sc as plsc

N = 10000
E = 320000
H = 128

_NC = 2
_NS = 16
_NW = _NC * _NS
_EW = E // _NW

_RPT = 624
_RTAIL = N - _NS * _RPT


def _leaky(x):
    return jnp.where(x >= 0, x, 0.1 * x)


def _dot(a, b):
    return jnp.dot(a, b, preferred_element_type=jnp.float32)


def _node_pre_body(x, w1, b1, w2, b2, wa, o_ref):
    h = _leaky(_dot(x[...], w1[...]) + b1[...])
    nf = _leaky(_dot(h, w2[...]) + b2[...])
    a = _dot(nf, wa[...])
    o_ref[...] = jnp.concatenate([a, jnp.zeros_like(a)], axis=1)


def _node_pre(node_feat, w1, b1, w2, b2, wa):
    bn = 1000
    return pl.pallas_call(
        _node_pre_body,
        grid=(N // bn,),
        in_specs=[
            pl.BlockSpec((bn, 128), lambda i: (i, 0)),
            pl.BlockSpec((128, 64), lambda i: (0, 0)),
            pl.BlockSpec((1, 64), lambda i: (0, 0)),
            pl.BlockSpec((64, 128), lambda i: (0, 0)),
            pl.BlockSpec((1, 128), lambda i: (0, 0)),
            pl.BlockSpec((128, 64), lambda i: (0, 0)),
        ],
        out_specs=pl.BlockSpec((bn, 128), lambda i: (i, 0)),
        out_shape=jax.ShapeDtypeStruct((N, 128), jnp.float32),
    )(node_feat, w1, b1, w2, b2, wa)


def _edge_fused_body(xt, g1, w1e, b1e, w2e, b2e, we1, be1, we2, be2,
                     wb1, bb1, w21, b21, wb2, bb2,
                     ef_ref, e2_ref, m1_ref):
    h = _leaky(jax.lax.dot_general(xt[...], w1e[...],
                                   (((0,), (0,)), ((), ())),
                                   preferred_element_type=jnp.float32)
               + b1e[...])
    ef = _leaky(_dot(h, w2e[...]) + b2e[...])
    ef_ref[...] = ef
    h2 = _leaky(_dot(ef, we1[...]) + be1[...])
    ef2 = _leaky(_dot(h2, we2[...]) + be2[...])
    e2_ref[...] = _dot(ef2, wb2[...]) + bb2[...]
    hm = _leaky(g1[...][:, :64] + _dot(ef, wb1[...]) + bb1[...])
    m1_ref[...] = _leaky(_dot(hm, w21[...]) + b21[...])


def _edge_fused(edge_feat_t, g1, w1e, b1e, w2e, b2e, we1, be1, we2, be2,
                wb1, bb1, w21, b21, wb2, bb2):
    be = 2560
    full = lambda r, c: pl.BlockSpec((r, c), lambda i: (0, 0))
    return pl.pallas_call(
        _edge_fused_body,
        grid=(E // be,),
        in_specs=[
            pl.BlockSpec((16, be), lambda i: (0, i)),
            pl.BlockSpec((be, 128), lambda i: (i, 0)),
            full(16, 64), full(1, 64), full(64, 128), full(1, 128),
            full(128, 64), full(1, 64), full(64, 128), full(1, 128),
            full(128, 64), full(1, 64), full(64, 128), full(1, 128),
            full(128, 64), full(1, 64),
        ],
        out_specs=[
            pl.BlockSpec((be, 128), lambda i: (i, 0)),
            pl.BlockSpec((be, 64), lambda i: (i, 0)),
            pl.BlockSpec((be, 128), lambda i: (i, 0)),
        ],
        out_shape=[
            jax.ShapeDtypeStruct((E, 128), jnp.float32),
            jax.ShapeDtypeStruct((E, 64), jnp.float32),
            jax.ShapeDtypeStruct((E, 128), jnp.float32),
        ],
    )(edge_feat_t, g1, w1e, b1e, w2e, b2e, we1, be1, we2, be2,
      wb1, bb1, w21, b21, wb2, bb2)


def _msg2_body(g2, e2, w22, b22, m_ref):
    hm = _leaky(g2[...][:, :64] + e2[...])
    m_ref[...] = _leaky(_dot(hm, w22[...]) + b22[...])


def _msg2(g2, e2, w22, b22):
    be = 2000
    return pl.pallas_call(
        _msg2_body,
        grid=(E // be,),
        in_specs=[
            pl.BlockSpec((be, 128), lambda i: (i, 0)),
            pl.BlockSpec((be, 64), lambda i: (i, 0)),
            pl.BlockSpec((64, 128), lambda i: (0, 0)),
            pl.BlockSpec((1, 128), lambda i: (0, 0)),
        ],
        out_specs=pl.BlockSpec((be, 128), lambda i: (i, 0)),
        out_shape=jax.ShapeDtypeStruct((E, 128), jnp.float32),
    )(g2, e2, w22, b22)


def _node_pre2_body(p, wa, o_ref):
    a = _dot(p[0] + p[1], wa[...])
    o_ref[...] = jnp.concatenate([a, jnp.zeros_like(a)], axis=1)


def _node_pre2(p, wa):
    bn = 1000
    return pl.pallas_call(
        _node_pre2_body,
        grid=(N // bn,),
        in_specs=[
            pl.BlockSpec((2, bn, 128), lambda i: (0, i, 0)),
            pl.BlockSpec((128, 64), lambda i: (0, 0)),
        ],
        out_specs=pl.BlockSpec((bn, 128), lambda i: (i, 0)),
        out_shape=jax.ShapeDtypeStruct((N, 128), jnp.float32),
    )(p, wa)


def _graph_body(xq, w, b, o_ref):
    k = pl.program_id(0)

    @pl.when(k == 0)
    def _():
        o_ref[...] = jnp.zeros_like(o_ref)

    x = xq[0:1, :] + xq[1:2, :]
    o_ref[...] += _dot(x, w[...])

    @pl.when(k == pl.num_programs(0) - 1)
    def _():
        o_ref[...] = _leaky(o_ref[...] + b[...])


def _graph_linear(q2, graph_W, graph_b):
    bk = 12800
    k_total = N * H
    return pl.pallas_call(
        _graph_body,
        grid=(k_total // bk,),
        in_specs=[
            pl.BlockSpec((2, bk), lambda k: (0, k)),
            pl.BlockSpec((bk, 128), lambda k: (k, 0)),
            pl.BlockSpec((1, 128), lambda k: (0, 0)),
        ],
        out_specs=pl.BlockSpec((1, 128), lambda k: (0, 0)),
        out_shape=jax.ShapeDtypeStruct((1, 128), jnp.float32),
    )(q2, graph_W, graph_b)


def _sc_gather(table, dst):
    mesh = plsc.VectorSubcoreMesh(core_axis_name="c", subcore_axis_name="s")
    chunk = 400
    nch = _EW // chunk

    @functools.partial(
        pl.kernel,
        out_type=jax.ShapeDtypeStruct((E, H), jnp.float32),
        mesh=mesh,
        scratch_types=[
            pltpu.VMEM((chunk,), jnp.int32),
            pltpu.VMEM((chunk,), jnp.int32),
            pltpu.VMEM((chunk, H), jnp.float32),
            pltpu.VMEM((chunk, H), jnp.float32),
            pltpu.SemaphoreType.DMA,
            pltpu.SemaphoreType.DMA,
        ],
    )
    def k(table_hbm, ei_hbm, out_hbm, idx0, idx1, rows0, rows1, sem0, sem1):
        wid = lax.axis_index("s") * _NC + lax.axis_index("c")
        base = wid * _EW
        rows = (rows0, rows1)
        idxs = (idx0, idx1)
        sems = (sem0, sem1)

        def body(g, carry):
            gbase = base + g * (chunk * 5)
            descs = [None, None]
            pltpu.sync_copy(ei_hbm.at[pl.ds(gbase, chunk)], idx0)
            descs[0] = pltpu.async_copy(table_hbm.at[idx0], rows0, sem0)
            for j in range(5):
                b, nb = j % 2, (j + 1) % 2
                if j + 1 < 5:
                    off = gbase + (j + 1) * chunk
                    pltpu.sync_copy(ei_hbm.at[pl.ds(off, chunk)], idxs[nb])
                    descs[nb] = pltpu.async_copy(table_hbm.at[idxs[nb]],
                                                 rows[nb], sems[nb])
                descs[b].wait()
                pltpu.sync_copy(rows[b],
                                out_hbm.at[pl.ds(gbase + j * chunk, chunk)])
            return carry

        lax.fori_loop(0, nch // 5, body, 0)

    return k(table, dst)


def _sc_scatter(m, dst):
    mesh = plsc.VectorSubcoreMesh(core_axis_name="c", subcore_axis_name="s")
    chunk = 200
    zrows = jnp.zeros((16, H), jnp.float32)

    @functools.partial(
        pl.kernel,
        out_type=jax.ShapeDtypeStruct((_NC, N, H), jnp.float32),
        mesh=mesh,
        scratch_types=[
            pltpu.VMEM((chunk,), jnp.int32),
            pltpu.VMEM((chunk, H), jnp.float32),
            pltpu.VMEM_SHARED((N, H), jnp.float32),
            pltpu.SemaphoreType.DMA,
        ],
    )
    def k(m_hbm, ei_hbm, z_hbm, out_hbm, idx0, rows0, acc_sh, sem0):
        c = lax.axis_index("c")
        s = lax.axis_index("s")
        wid = s * _NC + c
        def zbody(j, carry):
            pltpu.sync_copy(z_hbm, acc_sh.at[pl.ds(s * _RPT + j * 16, 16)])
            return carry

        lax.fori_loop(0, _RPT // 16, zbody, 0)

        @pl.when(s == _NS - 1)
        def _():
            pltpu.sync_copy(z_hbm, acc_sh.at[pl.ds(_NS * _RPT, _RTAIL)])

        plsc.subcore_barrier()

        base = wid * _EW

        def body(g, carry):
            off = base + g * chunk
            pltpu.sync_copy(ei_hbm.at[pl.ds(off, chunk)], idx0)
            pltpu.sync_copy(m_hbm.at[pl.ds(off, chunk)], rows0)
            pltpu.sync_copy(rows0, acc_sh.at[idx0], add=True)
            return carry

        lax.fori_loop(0, _EW // chunk, body, 0)
        plsc.subcore_barrier()
        pltpu.sync_copy(acc_sh.at[pl.ds(s * _RPT, _RPT)],
                        out_hbm.at[c, pl.ds(s * _RPT, _RPT)])

        @pl.when(s == _NS - 1)
        def _():
            pltpu.sync_copy(acc_sh.at[pl.ds(_NS * _RPT, _RTAIL)],
                            out_hbm.at[c, pl.ds(_NS * _RPT, _RTAIL)])

    return k(m, dst, zrows)


def kernel(node_feat, edge_feat, edge_index, mlp_node, mlp_edge,
           gnn1_mlp, gnn2_mlp, gnn2_mlp_edge, graph_W, graph_b):
    w1n, b1n, w2n, b2n = mlp_node
    w1e, b1e, w2e, b2e = mlp_edge
    w11, b11, w21, b21 = gnn1_mlp
    w12, b12, w22, b22 = gnn2_mlp
    we1, be1, we2, be2 = gnn2_mlp_edge

    r = lambda v: v.reshape(1, -1)
    wa1, wb1 = w11[:H], w11[H:]
    wa2, wb2 = w12[:H], w12[H:]

    dst = edge_index[1].astype(jnp.int32)

    a1 = _node_pre(node_feat, w1n, r(b1n), w2n, r(b2n), wa1)
    g1 = _sc_gather(a1, dst)

    ef, e2, m1 = _edge_fused(edge_feat.T, g1, w1e, r(b1e), w2e, r(b2e),
                             we1, r(be1), we2, r(be2),
                             wb1, r(b11), w21, r(b21), wb2, r(b12))
    p = _sc_scatter(m1, dst)

    a2 = _node_pre2(p, wa2)
    g2 = _sc_gather(a2, dst)
    m2 = _msg2(g2, e2, w22, r(b22))
    q = _sc_scatter(m2, dst)

    g = _graph_linear(q.reshape(_NC, N * H), graph_W, r(graph_b))
    return (g.reshape(H), ef)


# speedup vs baseline: 1.1178x; 1.0619x over previous
import functools

import jax
import jax.numpy as jnp
from jax import lax
from jax.experimental import pallas as pl
from jax.experimental.pallas import tpu as pltpu
from jax.experimental.pallas import tpu_sc as plsc

N = 10000
E = 320000
H = 128

_NC = 2
_NS = 16
_NW = _NC * _NS
_EW = E // _NW

_RPT = 624
_RTAIL = N - _NS * _RPT


def _leaky(x):
    return jnp.where(x >= 0, x, 0.1 * x)


def _dot(a, b):
    return jnp.dot(a, b, preferred_element_type=jnp.float32)


def _node_pre_body(x, w1, b1, w2, b2, wa, o_ref):
    h = _leaky(_dot(x[...], w1[...]) + b1[...])
    nf = _leaky(_dot(h, w2[...]) + b2[...])
    a = _dot(nf, wa[...])
    o_ref[...] = jnp.concatenate([a, jnp.zeros_like(a)], axis=1)


def _node_pre(node_feat, w1, b1, w2, b2, wa):
    bn = 1000
    return pl.pallas_call(
        _node_pre_body,
        grid=(N // bn,),
        in_specs=[
            pl.BlockSpec((bn, 128), lambda i: (i, 0)),
            pl.BlockSpec((128, 64), lambda i: (0, 0)),
            pl.BlockSpec((1, 64), lambda i: (0, 0)),
            pl.BlockSpec((64, 128), lambda i: (0, 0)),
            pl.BlockSpec((1, 128), lambda i: (0, 0)),
            pl.BlockSpec((128, 64), lambda i: (0, 0)),
        ],
        out_specs=pl.BlockSpec((bn, 128), lambda i: (i, 0)),
        out_shape=jax.ShapeDtypeStruct((N, 128), jnp.float32),
    )(node_feat, w1, b1, w2, b2, wa)


def _edge_fused_body(xt, g1, w1e, b1e, w2e, b2e, we1, be1, we2, be2,
                     wb1, bb1, w21, b21, wb2, bb2,
                     ef_ref, e2_ref, m1_ref):
    h = _leaky(jax.lax.dot_general(xt[...], w1e[...],
                                   (((0,), (0,)), ((), ())),
                                   preferred_element_type=jnp.float32)
               + b1e[...])
    ef = _leaky(_dot(h, w2e[...]) + b2e[...])
    ef_ref[...] = ef
    h2 = _leaky(_dot(ef, we1[...]) + be1[...])
    ef2 = _leaky(_dot(h2, we2[...]) + be2[...])
    e2_ref[...] = _dot(ef2, wb2[...]) + bb2[...]
    hm = _leaky(g1[...][:, :64] + _dot(ef, wb1[...]) + bb1[...])
    m1_ref[...] = _leaky(_dot(hm, w21[...]) + b21[...])


def _edge_fused(edge_feat_t, g1, w1e, b1e, w2e, b2e, we1, be1, we2, be2,
                wb1, bb1, w21, b21, wb2, bb2):
    be = 3200
    full = lambda r, c: pl.BlockSpec((r, c), lambda i: (0, 0))
    return pl.pallas_call(
        _edge_fused_body,
        grid=(E // be,),
        in_specs=[
            pl.BlockSpec((16, be), lambda i: (0, i)),
            pl.BlockSpec((be, 128), lambda i: (i, 0)),
            full(16, 64), full(1, 64), full(64, 128), full(1, 128),
            full(128, 64), full(1, 64), full(64, 128), full(1, 128),
            full(128, 64), full(1, 64), full(64, 128), full(1, 128),
            full(128, 64), full(1, 64),
        ],
        out_specs=[
            pl.BlockSpec((be, 128), lambda i: (i, 0)),
            pl.BlockSpec((be, 64), lambda i: (i, 0)),
            pl.BlockSpec((be, 128), lambda i: (i, 0)),
        ],
        out_shape=[
            jax.ShapeDtypeStruct((E, 128), jnp.float32),
            jax.ShapeDtypeStruct((E, 64), jnp.float32),
            jax.ShapeDtypeStruct((E, 128), jnp.float32),
        ],
    )(edge_feat_t, g1, w1e, b1e, w2e, b2e, we1, be1, we2, be2,
      wb1, bb1, w21, b21, wb2, bb2)


def _msg2_body(g2, e2, w22, b22, m_ref):
    hm = _leaky(g2[...][:, :64] + e2[...])
    m_ref[...] = _leaky(_dot(hm, w22[...]) + b22[...])


def _msg2(g2, e2, w22, b22):
    be = 4000
    return pl.pallas_call(
        _msg2_body,
        grid=(E // be,),
        in_specs=[
            pl.BlockSpec((be, 128), lambda i: (i, 0)),
            pl.BlockSpec((be, 64), lambda i: (i, 0)),
            pl.BlockSpec((64, 128), lambda i: (0, 0)),
            pl.BlockSpec((1, 128), lambda i: (0, 0)),
        ],
        out_specs=pl.BlockSpec((be, 128), lambda i: (i, 0)),
        out_shape=jax.ShapeDtypeStruct((E, 128), jnp.float32),
    )(g2, e2, w22, b22)


def _node_pre2_body(p, wa, o_ref):
    a = _dot(p[0] + p[1], wa[...])
    o_ref[...] = jnp.concatenate([a, jnp.zeros_like(a)], axis=1)


def _node_pre2(p, wa):
    bn = 1000
    return pl.pallas_call(
        _node_pre2_body,
        grid=(N // bn,),
        in_specs=[
            pl.BlockSpec((2, bn, 128), lambda i: (0, i, 0)),
            pl.BlockSpec((128, 64), lambda i: (0, 0)),
        ],
        out_specs=pl.BlockSpec((bn, 128), lambda i: (i, 0)),
        out_shape=jax.ShapeDtypeStruct((N, 128), jnp.float32),
    )(p, wa)


def _graph_body(xq, w, b, o_ref):
    k = pl.program_id(0)

    @pl.when(k == 0)
    def _():
        o_ref[...] = jnp.zeros_like(o_ref)

    x = xq[0:1, :] + xq[1:2, :]
    o_ref[...] += _dot(x, w[...])

    @pl.when(k == pl.num_programs(0) - 1)
    def _():
        o_ref[...] = _leaky(o_ref[...] + b[...])


def _graph_linear(q2, graph_W, graph_b):
    bk = 12800
    k_total = N * H
    return pl.pallas_call(
        _graph_body,
        grid=(k_total // bk,),
        in_specs=[
            pl.BlockSpec((2, bk), lambda k: (0, k)),
            pl.BlockSpec((bk, 128), lambda k: (k, 0)),
            pl.BlockSpec((1, 128), lambda k: (0, 0)),
        ],
        out_specs=pl.BlockSpec((1, 128), lambda k: (0, 0)),
        out_shape=jax.ShapeDtypeStruct((1, 128), jnp.float32),
    )(q2, graph_W, graph_b)


def _sc_gather(table, dst):
    mesh = plsc.VectorSubcoreMesh(core_axis_name="c", subcore_axis_name="s")
    chunk = 400
    nch = _EW // chunk

    @functools.partial(
        pl.kernel,
        out_type=jax.ShapeDtypeStruct((E, H), jnp.float32),
        mesh=mesh,
        scratch_types=[
            pltpu.VMEM((chunk,), jnp.int32),
            pltpu.VMEM((chunk,), jnp.int32),
            pltpu.VMEM((chunk, H), jnp.float32),
            pltpu.VMEM((chunk, H), jnp.float32),
            pltpu.SemaphoreType.DMA,
            pltpu.SemaphoreType.DMA,
        ],
    )
    def k(table_hbm, ei_hbm, out_hbm, idx0, idx1, rows0, rows1, sem0, sem1):
        wid = lax.axis_index("s") * _NC + lax.axis_index("c")
        base = wid * _EW
        rows = (rows0, rows1)
        idxs = (idx0, idx1)
        sems = (sem0, sem1)

        def body(g, carry):
            gbase = base + g * (chunk * 5)
            descs = [None, None]
            pltpu.sync_copy(ei_hbm.at[pl.ds(gbase, chunk)], idx0)
            descs[0] = pltpu.async_copy(table_hbm.at[idx0], rows0, sem0)
            for j in range(5):
                b, nb = j % 2, (j + 1) % 2
                if j + 1 < 5:
                    off = gbase + (j + 1) * chunk
                    pltpu.sync_copy(ei_hbm.at[pl.ds(off, chunk)], idxs[nb])
                    descs[nb] = pltpu.async_copy(table_hbm.at[idxs[nb]],
                                                 rows[nb], sems[nb])
                descs[b].wait()
                pltpu.sync_copy(rows[b],
                                out_hbm.at[pl.ds(gbase + j * chunk, chunk)])
            return carry

        lax.fori_loop(0, nch // 5, body, 0)

    return k(table, dst)


def _sc_scatter(m, dst):
    mesh = plsc.VectorSubcoreMesh(core_axis_name="c", subcore_axis_name="s")
    chunk, unroll = 80, 5
    outer = _EW // (chunk * unroll)
    zrows = jnp.zeros((16, H), jnp.float32)

    @functools.partial(
        pl.kernel,
        out_type=jax.ShapeDtypeStruct((_NC, N, H), jnp.float32),
        mesh=mesh,
        scratch_types=[
            pltpu.VMEM((chunk,), jnp.int32),
            pltpu.VMEM((chunk,), jnp.int32),
            pltpu.VMEM((chunk, H), jnp.float32),
            pltpu.VMEM((chunk, H), jnp.float32),
            pltpu.VMEM_SHARED((N, H), jnp.float32),
            pltpu.SemaphoreType.DMA,
            pltpu.SemaphoreType.DMA,
        ],
    )
    def k(m_hbm, ei_hbm, z_hbm, out_hbm, idx0, idx1, rows0, rows1, acc_sh,
          sem0, sem1):
        c = lax.axis_index("c")
        s = lax.axis_index("s")
        wid = s * _NC + c
        def zbody(j, carry):
            pltpu.sync_copy(z_hbm, acc_sh.at[pl.ds(s * _RPT + j * 16, 16)])
            return carry

        lax.fori_loop(0, _RPT // 16, zbody, 0)

        @pl.when(s == _NS - 1)
        def _():
            pltpu.sync_copy(z_hbm, acc_sh.at[pl.ds(_NS * _RPT, _RTAIL)])

        plsc.subcore_barrier()

        base = wid * _EW
        rows = (rows0, rows1)
        idxs = (idx0, idx1)
        sems = (sem0, sem1)

        def body(g, carry):
            gbase = base + g * (chunk * unroll)
            descs = [None, None]
            pltpu.sync_copy(ei_hbm.at[pl.ds(gbase, chunk)], idx0)
            descs[0] = pltpu.async_copy(m_hbm.at[pl.ds(gbase, chunk)],
                                        rows0, sem0)
            for j in range(unroll):
                b, nb = j % 2, (j + 1) % 2
                if j + 1 < unroll:
                    off = gbase + (j + 1) * chunk
                    pltpu.sync_copy(ei_hbm.at[pl.ds(off, chunk)], idxs[nb])
                    descs[nb] = pltpu.async_copy(m_hbm.at[pl.ds(off, chunk)],
                                                 rows[nb], sems[nb])
                descs[b].wait()
                pltpu.sync_copy(rows[b], acc_sh.at[idxs[b]], add=True)
            return carry

        lax.fori_loop(0, outer, body, 0)
        plsc.subcore_barrier()
        pltpu.sync_copy(acc_sh.at[pl.ds(s * _RPT, _RPT)],
                        out_hbm.at[c, pl.ds(s * _RPT, _RPT)])

        @pl.when(s == _NS - 1)
        def _():
            pltpu.sync_copy(acc_sh.at[pl.ds(_NS * _RPT, _RTAIL)],
                            out_hbm.at[c, pl.ds(_NS * _RPT, _RTAIL)])

    return k(m, dst, zrows)


def kernel(node_feat, edge_feat, edge_index, mlp_node, mlp_edge,
           gnn1_mlp, gnn2_mlp, gnn2_mlp_edge, graph_W, graph_b):
    w1n, b1n, w2n, b2n = mlp_node
    w1e, b1e, w2e, b2e = mlp_edge
    w11, b11, w21, b21 = gnn1_mlp
    w12, b12, w22, b22 = gnn2_mlp
    we1, be1, we2, be2 = gnn2_mlp_edge

    r = lambda v: v.reshape(1, -1)
    wa1, wb1 = w11[:H], w11[H:]
    wa2, wb2 = w12[:H], w12[H:]

    dst = edge_index[1].astype(jnp.int32)

    a1 = _node_pre(node_feat, w1n, r(b1n), w2n, r(b2n), wa1)
    g1 = _sc_gather(a1, dst)

    ef, e2, m1 = _edge_fused(edge_feat.T, g1, w1e, r(b1e), w2e, r(b2e),
                             we1, r(be1), we2, r(be2),
                             wb1, r(b11), w21, r(b21), wb2, r(b12))
    p = _sc_scatter(m1, dst)

    a2 = _node_pre2(p, wa2)
    g2 = _sc_gather(a2, dst)
    m2 = _msg2(g2, e2, w22, r(b22))
    q = _sc_scatter(m2, dst)

    g = _graph_linear(q.reshape(_NC, N * H), graph_W, r(graph_b))
    return (g.reshape(H), ef)


# speedup vs baseline: 1.1534x; 1.0318x over previous
import functools

import jax
import jax.numpy as jnp
from jax import lax
from jax.experimental import pallas as pl
from jax.experimental.pallas import tpu as pltpu
from jax.experimental.pallas import tpu_sc as plsc

N = 10000
E = 320000
H = 128

_NC = 2
_NS = 16
_NW = _NC * _NS
_EW = E // _NW

_RPT = 624
_RTAIL = N - _NS * _RPT


def _leaky(x):
    return jnp.where(x >= 0, x, 0.1 * x)


def _dot(a, b):
    return jnp.dot(a, b, preferred_element_type=jnp.float32)


def _node_pre_body(x, w1, b1, w2, b2, wa, o_ref):
    h = _leaky(_dot(x[...], w1[...]) + b1[...])
    nf = _leaky(_dot(h, w2[...]) + b2[...])
    a = _dot(nf, wa[...])
    o_ref[...] = jnp.concatenate([a, jnp.zeros_like(a)], axis=1)


def _node_pre(node_feat, w1, b1, w2, b2, wa):
    bn = 1000
    return pl.pallas_call(
        _node_pre_body,
        grid=(N // bn,),
        in_specs=[
            pl.BlockSpec((bn, 128), lambda i: (i, 0)),
            pl.BlockSpec((128, 64), lambda i: (0, 0)),
            pl.BlockSpec((1, 64), lambda i: (0, 0)),
            pl.BlockSpec((64, 128), lambda i: (0, 0)),
            pl.BlockSpec((1, 128), lambda i: (0, 0)),
            pl.BlockSpec((128, 64), lambda i: (0, 0)),
        ],
        out_specs=pl.BlockSpec((bn, 128), lambda i: (i, 0)),
        out_shape=jax.ShapeDtypeStruct((N, 128), jnp.float32),
    )(node_feat, w1, b1, w2, b2, wa)


def _edge_fused_body(xt, g1, w1e, b1e, w2e, b2e, we1, be1, we2, be2,
                     wb1, bb1, w21, b21, wb2, bb2,
                     ef_ref, e2_ref, m1_ref):
    h = _leaky(jax.lax.dot_general(xt[...], w1e[...],
                                   (((0,), (0,)), ((), ())),
                                   preferred_element_type=jnp.float32)
               + b1e[...])
    ef = _leaky(_dot(h, w2e[...]) + b2e[...])
    ef_ref[...] = ef
    h2 = _leaky(_dot(ef, we1[...]) + be1[...])
    ef2 = _leaky(_dot(h2, we2[...]) + be2[...])
    e2_ref[...] = _dot(ef2, wb2[...]) + bb2[...]
    hm = _leaky(g1[...][:, :64] + _dot(ef, wb1[...]) + bb1[...])
    m1_ref[...] = _leaky(_dot(hm, w21[...]) + b21[...])


def _edge_fused(edge_feat_t, g1, w1e, b1e, w2e, b2e, we1, be1, we2, be2,
                wb1, bb1, w21, b21, wb2, bb2):
    be = 6400
    full = lambda r, c: pl.BlockSpec((r, c), lambda i: (0, 0))
    return pl.pallas_call(
        _edge_fused_body,
        grid=(E // be,),
        in_specs=[
            pl.BlockSpec((16, be), lambda i: (0, i)),
            pl.BlockSpec((be, 128), lambda i: (i, 0)),
            full(16, 64), full(1, 64), full(64, 128), full(1, 128),
            full(128, 64), full(1, 64), full(64, 128), full(1, 128),
            full(128, 64), full(1, 64), full(64, 128), full(1, 128),
            full(128, 64), full(1, 64),
        ],
        out_specs=[
            pl.BlockSpec((be, 128), lambda i: (i, 0)),
            pl.BlockSpec((be, 64), lambda i: (i, 0)),
            pl.BlockSpec((be, 128), lambda i: (i, 0)),
        ],
        out_shape=[
            jax.ShapeDtypeStruct((E, 128), jnp.float32),
            jax.ShapeDtypeStruct((E, 64), jnp.float32),
            jax.ShapeDtypeStruct((E, 128), jnp.float32),
        ],
    )(edge_feat_t, g1, w1e, b1e, w2e, b2e, we1, be1, we2, be2,
      wb1, bb1, w21, b21, wb2, bb2)


def _msg2_body(g2, e2, w22, b22, m_ref):
    hm = _leaky(g2[...][:, :64] + e2[...])
    m_ref[...] = _leaky(_dot(hm, w22[...]) + b22[...])


def _msg2(g2, e2, w22, b22):
    be = 8000
    return pl.pallas_call(
        _msg2_body,
        grid=(E // be,),
        in_specs=[
            pl.BlockSpec((be, 128), lambda i: (i, 0)),
            pl.BlockSpec((be, 64), lambda i: (i, 0)),
            pl.BlockSpec((64, 128), lambda i: (0, 0)),
            pl.BlockSpec((1, 128), lambda i: (0, 0)),
        ],
        out_specs=pl.BlockSpec((be, 128), lambda i: (i, 0)),
        out_shape=jax.ShapeDtypeStruct((E, 128), jnp.float32),
    )(g2, e2, w22, b22)


def _node_pre2_body(p, wa, o_ref):
    a = _dot(p[0] + p[1], wa[...])
    o_ref[...] = jnp.concatenate([a, jnp.zeros_like(a)], axis=1)


def _node_pre2(p, wa):
    bn = 1000
    return pl.pallas_call(
        _node_pre2_body,
        grid=(N // bn,),
        in_specs=[
            pl.BlockSpec((2, bn, 128), lambda i: (0, i, 0)),
            pl.BlockSpec((128, 64), lambda i: (0, 0)),
        ],
        out_specs=pl.BlockSpec((bn, 128), lambda i: (i, 0)),
        out_shape=jax.ShapeDtypeStruct((N, 128), jnp.float32),
    )(p, wa)


def _graph_body(xq, w, b, o_ref):
    k = pl.program_id(0)

    @pl.when(k == 0)
    def _():
        o_ref[...] = jnp.zeros_like(o_ref)

    x = xq[0:1, :] + xq[1:2, :]
    o_ref[...] += _dot(x, w[...])

    @pl.when(k == pl.num_programs(0) - 1)
    def _():
        o_ref[...] = _leaky(o_ref[...] + b[...])


def _graph_linear(q2, graph_W, graph_b):
    bk = 25600
    k_total = N * H
    return pl.pallas_call(
        _graph_body,
        grid=(k_total // bk,),
        in_specs=[
            pl.BlockSpec((2, bk), lambda k: (0, k)),
            pl.BlockSpec((bk, 128), lambda k: (k, 0)),
            pl.BlockSpec((1, 128), lambda k: (0, 0)),
        ],
        out_specs=pl.BlockSpec((1, 128), lambda k: (0, 0)),
        out_shape=jax.ShapeDtypeStruct((1, 128), jnp.float32),
    )(q2, graph_W, graph_b)


def _sc_gather(table, dst):
    mesh = plsc.VectorSubcoreMesh(core_axis_name="c", subcore_axis_name="s")
    chunk = 400
    nch = _EW // chunk

    @functools.partial(
        pl.kernel,
        out_type=jax.ShapeDtypeStruct((E, H), jnp.float32),
        mesh=mesh,
        scratch_types=[
            pltpu.VMEM((chunk,), jnp.int32),
            pltpu.VMEM((chunk,), jnp.int32),
            pltpu.VMEM((chunk, H), jnp.float32),
            pltpu.VMEM((chunk, H), jnp.float32),
            pltpu.SemaphoreType.DMA,
            pltpu.SemaphoreType.DMA,
        ],
    )
    def k(table_hbm, ei_hbm, out_hbm, idx0, idx1, rows0, rows1, sem0, sem1):
        wid = lax.axis_index("s") * _NC + lax.axis_index("c")
        base = wid * _EW
        rows = (rows0, rows1)
        idxs = (idx0, idx1)
        sems = (sem0, sem1)

        def body(g, carry):
            gbase = base + g * (chunk * 5)
            descs = [None, None]
            pltpu.sync_copy(ei_hbm.at[pl.ds(gbase, chunk)], idx0)
            descs[0] = pltpu.async_copy(table_hbm.at[idx0], rows0, sem0)
            for j in range(5):
                b, nb = j % 2, (j + 1) % 2
                if j + 1 < 5:
                    off = gbase + (j + 1) * chunk
                    pltpu.sync_copy(ei_hbm.at[pl.ds(off, chunk)], idxs[nb])
                    descs[nb] = pltpu.async_copy(table_hbm.at[idxs[nb]],
                                                 rows[nb], sems[nb])
                descs[b].wait()
                pltpu.sync_copy(rows[b],
                                out_hbm.at[pl.ds(gbase + j * chunk, chunk)])
            return carry

        lax.fori_loop(0, nch // 5, body, 0)

    return k(table, dst)


def _sc_scatter(m, dst):
    mesh = plsc.VectorSubcoreMesh(core_axis_name="c", subcore_axis_name="s")
    chunk, unroll = 80, 5
    outer = _EW // (chunk * unroll)
    zrows = jnp.zeros((16, H), jnp.float32)

    @functools.partial(
        pl.kernel,
        out_type=jax.ShapeDtypeStruct((_NC, N, H), jnp.float32),
        mesh=mesh,
        scratch_types=[
            pltpu.VMEM((chunk,), jnp.int32),
            pltpu.VMEM((chunk,), jnp.int32),
            pltpu.VMEM((chunk, H), jnp.float32),
            pltpu.VMEM((chunk, H), jnp.float32),
            pltpu.VMEM_SHARED((N, H), jnp.float32),
            pltpu.SemaphoreType.DMA,
            pltpu.SemaphoreType.DMA,
        ],
    )
    def k(m_hbm, ei_hbm, z_hbm, out_hbm, idx0, idx1, rows0, rows1, acc_sh,
          sem0, sem1):
        c = lax.axis_index("c")
        s = lax.axis_index("s")
        wid = s * _NC + c
        def zbody(j, carry):
            pltpu.sync_copy(z_hbm, acc_sh.at[pl.ds(s * _RPT + j * 16, 16)])
            return carry

        lax.fori_loop(0, _RPT // 16, zbody, 0)

        @pl.when(s == _NS - 1)
        def _():
            pltpu.sync_copy(z_hbm, acc_sh.at[pl.ds(_NS * _RPT, _RTAIL)])

        plsc.subcore_barrier()

        base = wid * _EW
        rows = (rows0, rows1)
        idxs = (idx0, idx1)
        sems = (sem0, sem1)

        def body(g, carry):
            gbase = base + g * (chunk * unroll)
            descs = [None, None]
            pltpu.sync_copy(ei_hbm.at[pl.ds(gbase, chunk)], idx0)
            descs[0] = pltpu.async_copy(m_hbm.at[pl.ds(gbase, chunk)],
                                        rows0, sem0)
            for j in range(unroll):
                b, nb = j % 2, (j + 1) % 2
                if j + 1 < unroll:
                    off = gbase + (j + 1) * chunk
                    pltpu.sync_copy(ei_hbm.at[pl.ds(off, chunk)], idxs[nb])
                    descs[nb] = pltpu.async_copy(m_hbm.at[pl.ds(off, chunk)],
                                                 rows[nb], sems[nb])
                descs[b].wait()
                pltpu.sync_copy(rows[b], acc_sh.at[idxs[b]], add=True)
            return carry

        lax.fori_loop(0, outer, body, 0)
        plsc.subcore_barrier()
        pltpu.sync_copy(acc_sh.at[pl.ds(s * _RPT, _RPT)],
                        out_hbm.at[c, pl.ds(s * _RPT, _RPT)])

        @pl.when(s == _NS - 1)
        def _():
            pltpu.sync_copy(acc_sh.at[pl.ds(_NS * _RPT, _RTAIL)],
                            out_hbm.at[c, pl.ds(_NS * _RPT, _RTAIL)])

    return k(m, dst, zrows)


def kernel(node_feat, edge_feat, edge_index, mlp_node, mlp_edge,
           gnn1_mlp, gnn2_mlp, gnn2_mlp_edge, graph_W, graph_b):
    w1n, b1n, w2n, b2n = mlp_node
    w1e, b1e, w2e, b2e = mlp_edge
    w11, b11, w21, b21 = gnn1_mlp
    w12, b12, w22, b22 = gnn2_mlp
    we1, be1, we2, be2 = gnn2_mlp_edge

    r = lambda v: v.reshape(1, -1)
    wa1, wb1 = w11[:H], w11[H:]
    wa2, wb2 = w12[:H], w12[H:]

    dst = edge_index[1].astype(jnp.int32)

    a1 = _node_pre(node_feat, w1n, r(b1n), w2n, r(b2n), wa1)
    g1 = _sc_gather(a1, dst)

    ef, e2, m1 = _edge_fused(edge_feat.T, g1, w1e, r(b1e), w2e, r(b2e),
                             we1, r(be1), we2, r(be2),
                             wb1, r(b11), w21, r(b21), wb2, r(b12))
    p = _sc_scatter(m1, dst)

    a2 = _node_pre2(p, wa2)
    g2 = _sc_gather(a2, dst)
    m2 = _msg2(g2, e2, w22, r(b22))
    q = _sc_scatter(m2, dst)

    g = _graph_linear(q.reshape(_NC, N * H), graph_W, r(graph_b))
    return (g.reshape(H), ef)


# speedup vs baseline: 1.1574x; 1.0034x over previous
import functools

import jax
import jax.numpy as jnp
from jax import lax
from jax.experimental import pallas as pl
from jax.experimental.pallas import tpu as pltpu
from jax.experimental.pallas import tpu_sc as plsc

N = 10000
E = 320000
H = 128

_NC = 2
_NS = 16
_NW = _NC * _NS
_EW = E // _NW

_RPT = 624
_RTAIL = N - _NS * _RPT


def _leaky(x):
    return jnp.where(x >= 0, x, 0.1 * x)


def _dot(a, b):
    return jnp.dot(a, b, preferred_element_type=jnp.float32)


def _node_pre_body(x, w1, b1, w2, b2, wa, o_ref):
    h = _leaky(_dot(x[...], w1[...]) + b1[...])
    nf = _leaky(_dot(h, w2[...]) + b2[...])
    a = _dot(nf, wa[...])
    o_ref[...] = jnp.concatenate([a, jnp.zeros_like(a)], axis=1)


def _node_pre(node_feat, w1, b1, w2, b2, wa):
    bn = 1000
    return pl.pallas_call(
        _node_pre_body,
        grid=(N // bn,),
        in_specs=[
            pl.BlockSpec((bn, 128), lambda i: (i, 0)),
            pl.BlockSpec((128, 64), lambda i: (0, 0)),
            pl.BlockSpec((1, 64), lambda i: (0, 0)),
            pl.BlockSpec((64, 128), lambda i: (0, 0)),
            pl.BlockSpec((1, 128), lambda i: (0, 0)),
            pl.BlockSpec((128, 64), lambda i: (0, 0)),
        ],
        out_specs=pl.BlockSpec((bn, 128), lambda i: (i, 0)),
        out_shape=jax.ShapeDtypeStruct((N, 128), jnp.float32),
    )(node_feat, w1, b1, w2, b2, wa)


def _edge_fused_body(xt, g1, w1e, b1e, w2e, b2e, we1, be1, we2, be2,
                     wb1, bb1, w21, b21, wb2, bb2,
                     ef_ref, e2_ref, m1_ref):
    h = _leaky(jax.lax.dot_general(xt[...], w1e[...],
                                   (((0,), (0,)), ((), ())),
                                   preferred_element_type=jnp.float32)
               + b1e[...])
    ef = _leaky(_dot(h, w2e[...]) + b2e[...])
    ef_ref[...] = ef
    h2 = _leaky(_dot(ef, we1[...]) + be1[...])
    ef2 = _leaky(_dot(h2, we2[...]) + be2[...])
    e2_ref[...] = _dot(ef2, wb2[...]) + bb2[...]
    hm = _leaky(g1[...][:, :64] + _dot(ef, wb1[...]) + bb1[...])
    m1_ref[...] = _leaky(_dot(hm, w21[...]) + b21[...])


def _edge_fused(edge_feat_t, g1, w1e, b1e, w2e, b2e, we1, be1, we2, be2,
                wb1, bb1, w21, b21, wb2, bb2):
    be = 12800
    full = lambda r, c: pl.BlockSpec((r, c), lambda i: (0, 0))
    return pl.pallas_call(
        _edge_fused_body,
        grid=(E // be,),
        in_specs=[
            pl.BlockSpec((16, be), lambda i: (0, i)),
            pl.BlockSpec((be, 128), lambda i: (i, 0)),
            full(16, 64), full(1, 64), full(64, 128), full(1, 128),
            full(128, 64), full(1, 64), full(64, 128), full(1, 128),
            full(128, 64), full(1, 64), full(64, 128), full(1, 128),
            full(128, 64), full(1, 64),
        ],
        out_specs=[
            pl.BlockSpec((be, 128), lambda i: (i, 0)),
            pl.BlockSpec((be, 64), lambda i: (i, 0)),
            pl.BlockSpec((be, 128), lambda i: (i, 0)),
        ],
        out_shape=[
            jax.ShapeDtypeStruct((E, 128), jnp.float32),
            jax.ShapeDtypeStruct((E, 64), jnp.float32),
            jax.ShapeDtypeStruct((E, 128), jnp.float32),
        ],
    )(edge_feat_t, g1, w1e, b1e, w2e, b2e, we1, be1, we2, be2,
      wb1, bb1, w21, b21, wb2, bb2)


def _msg2_body(g2, e2, w22, b22, m_ref):
    hm = _leaky(g2[...][:, :64] + e2[...])
    m_ref[...] = _leaky(_dot(hm, w22[...]) + b22[...])


def _msg2(g2, e2, w22, b22):
    be = 16000
    return pl.pallas_call(
        _msg2_body,
        grid=(E // be,),
        in_specs=[
            pl.BlockSpec((be, 128), lambda i: (i, 0)),
            pl.BlockSpec((be, 64), lambda i: (i, 0)),
            pl.BlockSpec((64, 128), lambda i: (0, 0)),
            pl.BlockSpec((1, 128), lambda i: (0, 0)),
        ],
        out_specs=pl.BlockSpec((be, 128), lambda i: (i, 0)),
        out_shape=jax.ShapeDtypeStruct((E, 128), jnp.float32),
    )(g2, e2, w22, b22)


def _node_pre2_body(p, wa, o_ref):
    a = _dot(p[0] + p[1], wa[...])
    o_ref[...] = jnp.concatenate([a, jnp.zeros_like(a)], axis=1)


def _node_pre2(p, wa):
    bn = 1000
    return pl.pallas_call(
        _node_pre2_body,
        grid=(N // bn,),
        in_specs=[
            pl.BlockSpec((2, bn, 128), lambda i: (0, i, 0)),
            pl.BlockSpec((128, 64), lambda i: (0, 0)),
        ],
        out_specs=pl.BlockSpec((bn, 128), lambda i: (i, 0)),
        out_shape=jax.ShapeDtypeStruct((N, 128), jnp.float32),
    )(p, wa)


def _graph_body(xq, w, b, o_ref):
    k = pl.program_id(0)

    @pl.when(k == 0)
    def _():
        o_ref[...] = jnp.zeros_like(o_ref)

    x = xq[0:1, :] + xq[1:2, :]
    o_ref[...] += _dot(x, w[...])

    @pl.when(k == pl.num_programs(0) - 1)
    def _():
        o_ref[...] = _leaky(o_ref[...] + b[...])


def _graph_linear(q2, graph_W, graph_b):
    bk = 25600
    k_total = N * H
    return pl.pallas_call(
        _graph_body,
        grid=(k_total // bk,),
        in_specs=[
            pl.BlockSpec((2, bk), lambda k: (0, k)),
            pl.BlockSpec((bk, 128), lambda k: (k, 0)),
            pl.BlockSpec((1, 128), lambda k: (0, 0)),
        ],
        out_specs=pl.BlockSpec((1, 128), lambda k: (0, 0)),
        out_shape=jax.ShapeDtypeStruct((1, 128), jnp.float32),
    )(q2, graph_W, graph_b)


def _sc_gather(table, dst):
    mesh = plsc.VectorSubcoreMesh(core_axis_name="c", subcore_axis_name="s")
    chunk = 400
    nch = _EW // chunk

    @functools.partial(
        pl.kernel,
        out_type=jax.ShapeDtypeStruct((E, H), jnp.float32),
        mesh=mesh,
        scratch_types=[
            pltpu.VMEM((chunk,), jnp.int32),
            pltpu.VMEM((chunk,), jnp.int32),
            pltpu.VMEM((chunk, H), jnp.float32),
            pltpu.VMEM((chunk, H), jnp.float32),
            pltpu.SemaphoreType.DMA,
            pltpu.SemaphoreType.DMA,
        ],
    )
    def k(table_hbm, ei_hbm, out_hbm, idx0, idx1, rows0, rows1, sem0, sem1):
        wid = lax.axis_index("s") * _NC + lax.axis_index("c")
        base = wid * _EW
        rows = (rows0, rows1)
        idxs = (idx0, idx1)
        sems = (sem0, sem1)

        def body(g, carry):
            gbase = base + g * (chunk * 5)
            descs = [None, None]
            pltpu.sync_copy(ei_hbm.at[pl.ds(gbase, chunk)], idx0)
            descs[0] = pltpu.async_copy(table_hbm.at[idx0], rows0, sem0)
            for j in range(5):
                b, nb = j % 2, (j + 1) % 2
                if j + 1 < 5:
                    off = gbase + (j + 1) * chunk
                    pltpu.sync_copy(ei_hbm.at[pl.ds(off, chunk)], idxs[nb])
                    descs[nb] = pltpu.async_copy(table_hbm.at[idxs[nb]],
                                                 rows[nb], sems[nb])
                descs[b].wait()
                pltpu.sync_copy(rows[b],
                                out_hbm.at[pl.ds(gbase + j * chunk, chunk)])
            return carry

        lax.fori_loop(0, nch // 5, body, 0)

    return k(table, dst)


def _sc_scatter(m, dst):
    mesh = plsc.VectorSubcoreMesh(core_axis_name="c", subcore_axis_name="s")
    chunk, unroll = 80, 5
    outer = _EW // (chunk * unroll)
    zrows = jnp.zeros((16, H), jnp.float32)

    @functools.partial(
        pl.kernel,
        out_type=jax.ShapeDtypeStruct((_NC, N, H), jnp.float32),
        mesh=mesh,
        scratch_types=[
            pltpu.VMEM((chunk,), jnp.int32),
            pltpu.VMEM((chunk,), jnp.int32),
            pltpu.VMEM((chunk, H), jnp.float32),
            pltpu.VMEM((chunk, H), jnp.float32),
            pltpu.VMEM_SHARED((N, H), jnp.float32),
            pltpu.SemaphoreType.DMA,
            pltpu.SemaphoreType.DMA,
        ],
    )
    def k(m_hbm, ei_hbm, z_hbm, out_hbm, idx0, idx1, rows0, rows1, acc_sh,
          sem0, sem1):
        c = lax.axis_index("c")
        s = lax.axis_index("s")
        wid = s * _NC + c
        def zbody(j, carry):
            pltpu.sync_copy(z_hbm, acc_sh.at[pl.ds(s * _RPT + j * 16, 16)])
            return carry

        lax.fori_loop(0, _RPT // 16, zbody, 0)

        @pl.when(s == _NS - 1)
        def _():
            pltpu.sync_copy(z_hbm, acc_sh.at[pl.ds(_NS * _RPT, _RTAIL)])

        plsc.subcore_barrier()

        base = wid * _EW
        rows = (rows0, rows1)
        idxs = (idx0, idx1)
        sems = (sem0, sem1)

        def body(g, carry):
            gbase = base + g * (chunk * unroll)
            descs = [None, None]
            pltpu.sync_copy(ei_hbm.at[pl.ds(gbase, chunk)], idx0)
            descs[0] = pltpu.async_copy(m_hbm.at[pl.ds(gbase, chunk)],
                                        rows0, sem0)
            for j in range(unroll):
                b, nb = j % 2, (j + 1) % 2
                if j + 1 < unroll:
                    off = gbase + (j + 1) * chunk
                    pltpu.sync_copy(ei_hbm.at[pl.ds(off, chunk)], idxs[nb])
                    descs[nb] = pltpu.async_copy(m_hbm.at[pl.ds(off, chunk)],
                                                 rows[nb], sems[nb])
                descs[b].wait()
                pltpu.sync_copy(rows[b], acc_sh.at[idxs[b]], add=True)
            return carry

        lax.fori_loop(0, outer, body, 0)
        plsc.subcore_barrier()
        pltpu.sync_copy(acc_sh.at[pl.ds(s * _RPT, _RPT)],
                        out_hbm.at[c, pl.ds(s * _RPT, _RPT)])

        @pl.when(s == _NS - 1)
        def _():
            pltpu.sync_copy(acc_sh.at[pl.ds(_NS * _RPT, _RTAIL)],
                            out_hbm.at[c, pl.ds(_NS * _RPT, _RTAIL)])

    return k(m, dst, zrows)


def kernel(node_feat, edge_feat, edge_index, mlp_node, mlp_edge,
           gnn1_mlp, gnn2_mlp, gnn2_mlp_edge, graph_W, graph_b):
    w1n, b1n, w2n, b2n = mlp_node
    w1e, b1e, w2e, b2e = mlp_edge
    w11, b11, w21, b21 = gnn1_mlp
    w12, b12, w22, b22 = gnn2_mlp
    we1, be1, we2, be2 = gnn2_mlp_edge

    r = lambda v: v.reshape(1, -1)
    wa1, wb1 = w11[:H], w11[H:]
    wa2, wb2 = w12[:H], w12[H:]

    dst = edge_index[1].astype(jnp.int32)

    a1 = _node_pre(node_feat, w1n, r(b1n), w2n, r(b2n), wa1)
    g1 = _sc_gather(a1, dst)

    ef, e2, m1 = _edge_fused(edge_feat.T, g1, w1e, r(b1e), w2e, r(b2e),
                             we1, r(be1), we2, r(be2),
                             wb1, r(b11), w21, r(b21), wb2, r(b12))
    p = _sc_scatter(m1, dst)

    a2 = _node_pre2(p, wa2)
    g2 = _sc_gather(a2, dst)
    m2 = _msg2(g2, e2, w22, r(b22))
    q = _sc_scatter(m2, dst)

    g = _graph_linear(q.reshape(_NC, N * H), graph_W, r(graph_b))
    return (g.reshape(H), ef)


# speedup vs baseline: 1.1892x; 1.0275x over previous
import functools

import jax
import jax.numpy as jnp
from jax import lax
from jax.experimental import pallas as pl
from jax.experimental.pallas import tpu as pltpu
from jax.experimental.pallas import tpu_sc as plsc

N = 10000
E = 320000
H = 128

_NC = 2
_NS = 16
_NW = _NC * _NS
_EW = E // _NW

_RPT = 624
_RTAIL = N - _NS * _RPT


def _leaky(x):
    return jnp.where(x >= 0, x, 0.1 * x)


def _dot(a, b):
    return jnp.dot(a, b, preferred_element_type=jnp.float32)


def _node_pre_body(x, w1, b1, w2, b2, wa, o_ref):
    h = _leaky(_dot(x[...], w1[...]) + b1[...])
    nf = _leaky(_dot(h, w2[...]) + b2[...])
    a = _dot(nf, wa[...])
    o_ref[...] = jnp.concatenate([a, jnp.zeros_like(a)], axis=1)


def _node_pre(node_feat, w1, b1, w2, b2, wa):
    bn = 1000
    return pl.pallas_call(
        _node_pre_body,
        grid=(N // bn,),
        in_specs=[
            pl.BlockSpec((bn, 128), lambda i: (i, 0)),
            pl.BlockSpec((128, 64), lambda i: (0, 0)),
            pl.BlockSpec((1, 64), lambda i: (0, 0)),
            pl.BlockSpec((64, 128), lambda i: (0, 0)),
            pl.BlockSpec((1, 128), lambda i: (0, 0)),
            pl.BlockSpec((128, 64), lambda i: (0, 0)),
        ],
        out_specs=pl.BlockSpec((bn, 128), lambda i: (i, 0)),
        out_shape=jax.ShapeDtypeStruct((N, 128), jnp.float32),
    )(node_feat, w1, b1, w2, b2, wa)


def _edge_fused_body(xt, g1, w1e, b1e, w2e, b2e, we1, be1, we2, be2,
                     wb1, bb1, w21, b21, wb2, bb2,
                     ef_ref, e2_ref, m1_ref):
    h = _leaky(jax.lax.dot_general(xt[...], w1e[...],
                                   (((0,), (0,)), ((), ())),
                                   preferred_element_type=jnp.float32)
               + b1e[...])
    ef = _leaky(_dot(h, w2e[...]) + b2e[...])
    ef_ref[...] = ef
    h2 = _leaky(_dot(ef, we1[...]) + be1[...])
    ef2 = _leaky(_dot(h2, we2[...]) + be2[...])
    e2_ref[...] = _dot(ef2, wb2[...]) + bb2[...]
    hm = _leaky(g1[...][:, :64] + _dot(ef, wb1[...]) + bb1[...])
    m1_ref[...] = _leaky(_dot(hm, w21[...]) + b21[...])


def _edge_fused(edge_feat_t, g1, w1e, b1e, w2e, b2e, we1, be1, we2, be2,
                wb1, bb1, w21, b21, wb2, bb2):
    be = 12800
    full = lambda r, c: pl.BlockSpec((r, c), lambda i: (0, 0))
    return pl.pallas_call(
        _edge_fused_body,
        grid=(E // be,),
        in_specs=[
            pl.BlockSpec((16, be), lambda i: (0, i)),
            pl.BlockSpec((be, 128), lambda i: (i, 0)),
            full(16, 64), full(1, 64), full(64, 128), full(1, 128),
            full(128, 64), full(1, 64), full(64, 128), full(1, 128),
            full(128, 64), full(1, 64), full(64, 128), full(1, 128),
            full(128, 64), full(1, 64),
        ],
        out_specs=[
            pl.BlockSpec((be, 128), lambda i: (i, 0)),
            pl.BlockSpec((be, 64), lambda i: (i, 0)),
            pl.BlockSpec((be, 128), lambda i: (i, 0)),
        ],
        out_shape=[
            jax.ShapeDtypeStruct((E, 128), jnp.float32),
            jax.ShapeDtypeStruct((E, 64), jnp.float32),
            jax.ShapeDtypeStruct((E, 128), jnp.float32),
        ],
    )(edge_feat_t, g1, w1e, b1e, w2e, b2e, we1, be1, we2, be2,
      wb1, bb1, w21, b21, wb2, bb2)


def _msg2_body(g2, e2, w22, b22, m_ref):
    hm = _leaky(g2[...][:, :64] + e2[...])
    m_ref[...] = _leaky(_dot(hm, w22[...]) + b22[...])


def _msg2(g2, e2, w22, b22):
    be = 16000
    return pl.pallas_call(
        _msg2_body,
        grid=(E // be,),
        in_specs=[
            pl.BlockSpec((be, 128), lambda i: (i, 0)),
            pl.BlockSpec((be, 64), lambda i: (i, 0)),
            pl.BlockSpec((64, 128), lambda i: (0, 0)),
            pl.BlockSpec((1, 128), lambda i: (0, 0)),
        ],
        out_specs=pl.BlockSpec((be, 128), lambda i: (i, 0)),
        out_shape=jax.ShapeDtypeStruct((E, 128), jnp.float32),
    )(g2, e2, w22, b22)


def _node_pre2_body(p, wa, o_ref):
    a = _dot(p[0] + p[1], wa[...])
    o_ref[...] = jnp.concatenate([a, jnp.zeros_like(a)], axis=1)


def _node_pre2(p, wa):
    bn = 1000
    return pl.pallas_call(
        _node_pre2_body,
        grid=(N // bn,),
        in_specs=[
            pl.BlockSpec((2, bn, 128), lambda i: (0, i, 0)),
            pl.BlockSpec((128, 64), lambda i: (0, 0)),
        ],
        out_specs=pl.BlockSpec((bn, 128), lambda i: (i, 0)),
        out_shape=jax.ShapeDtypeStruct((N, 128), jnp.float32),
    )(p, wa)


def _graph_body(xq, w, b, o_ref):
    k = pl.program_id(0)

    @pl.when(k == 0)
    def _():
        o_ref[...] = jnp.zeros_like(o_ref)

    x = xq[0:1, :] + xq[1:2, :]
    o_ref[...] += _dot(x, w[...])

    @pl.when(k == pl.num_programs(0) - 1)
    def _():
        o_ref[...] = _leaky(o_ref[...] + b[...])


def _graph_linear(q2, graph_W, graph_b):
    bk = 25600
    k_total = N * H
    return pl.pallas_call(
        _graph_body,
        grid=(k_total // bk,),
        in_specs=[
            pl.BlockSpec((2, bk), lambda k: (0, k)),
            pl.BlockSpec((bk, 128), lambda k: (k, 0)),
            pl.BlockSpec((1, 128), lambda k: (0, 0)),
        ],
        out_specs=pl.BlockSpec((1, 128), lambda k: (0, 0)),
        out_shape=jax.ShapeDtypeStruct((1, 128), jnp.float32),
    )(q2, graph_W, graph_b)


def _sc_gather(table, dst):
    mesh = plsc.VectorSubcoreMesh(core_axis_name="c", subcore_axis_name="s")
    chunk = 400
    nch = _EW // chunk

    @functools.partial(
        pl.kernel,
        out_type=jax.ShapeDtypeStruct((E, H), jnp.float32),
        mesh=mesh,
        scratch_types=[
            pltpu.VMEM((chunk,), jnp.int32),
            pltpu.VMEM((chunk,), jnp.int32),
            pltpu.VMEM((chunk, H), jnp.float32),
            pltpu.VMEM((chunk, H), jnp.float32),
            pltpu.SemaphoreType.DMA,
            pltpu.SemaphoreType.DMA,
        ],
    )
    def k(table_hbm, ei_hbm, out_hbm, idx0, idx1, rows0, rows1, sem0, sem1):
        wid = lax.axis_index("s") * _NC + lax.axis_index("c")
        base = wid * _EW
        rows = (rows0, rows1)
        idxs = (idx0, idx1)
        sems = (sem0, sem1)

        descs = [None, None]
        pltpu.sync_copy(ei_hbm.at[pl.ds(base, chunk)], idx0)
        descs[0] = pltpu.async_copy(table_hbm.at[idx0], rows0, sem0)
        for j in range(nch):
            b, nb = j % 2, (j + 1) % 2
            if j + 1 < nch:
                off = base + (j + 1) * chunk
                pltpu.sync_copy(ei_hbm.at[pl.ds(off, chunk)], idxs[nb])
                descs[nb] = pltpu.async_copy(table_hbm.at[idxs[nb]],
                                             rows[nb], sems[nb])
            descs[b].wait()
            pltpu.sync_copy(rows[b],
                            out_hbm.at[pl.ds(base + j * chunk, chunk)])

    return k(table, dst)


def _sc_scatter(m, dst):
    mesh = plsc.VectorSubcoreMesh(core_axis_name="c", subcore_axis_name="s")
    chunk, unroll = 80, 25
    outer = _EW // (chunk * unroll)
    zrows = jnp.zeros((16, H), jnp.float32)

    @functools.partial(
        pl.kernel,
        out_type=jax.ShapeDtypeStruct((_NC, N, H), jnp.float32),
        mesh=mesh,
        scratch_types=[
            pltpu.VMEM((chunk,), jnp.int32),
            pltpu.VMEM((chunk,), jnp.int32),
            pltpu.VMEM((chunk, H), jnp.float32),
            pltpu.VMEM((chunk, H), jnp.float32),
            pltpu.VMEM_SHARED((N, H), jnp.float32),
            pltpu.SemaphoreType.DMA,
            pltpu.SemaphoreType.DMA,
        ],
    )
    def k(m_hbm, ei_hbm, z_hbm, out_hbm, idx0, idx1, rows0, rows1, acc_sh,
          sem0, sem1):
        c = lax.axis_index("c")
        s = lax.axis_index("s")
        wid = s * _NC + c
        def zbody(j, carry):
            pltpu.sync_copy(z_hbm, acc_sh.at[pl.ds(s * _RPT + j * 16, 16)])
            return carry

        lax.fori_loop(0, _RPT // 16, zbody, 0)

        @pl.when(s == _NS - 1)
        def _():
            pltpu.sync_copy(z_hbm, acc_sh.at[pl.ds(_NS * _RPT, _RTAIL)])

        plsc.subcore_barrier()

        base = wid * _EW
        rows = (rows0, rows1)
        idxs = (idx0, idx1)
        sems = (sem0, sem1)

        def body(g, carry):
            gbase = base + g * (chunk * unroll)
            descs = [None, None]
            pltpu.sync_copy(ei_hbm.at[pl.ds(gbase, chunk)], idx0)
            descs[0] = pltpu.async_copy(m_hbm.at[pl.ds(gbase, chunk)],
                                        rows0, sem0)
            for j in range(unroll):
                b, nb = j % 2, (j + 1) % 2
                if j + 1 < unroll:
                    off = gbase + (j + 1) * chunk
                    pltpu.sync_copy(ei_hbm.at[pl.ds(off, chunk)], idxs[nb])
                    descs[nb] = pltpu.async_copy(m_hbm.at[pl.ds(off, chunk)],
                                                 rows[nb], sems[nb])
                descs[b].wait()
                pltpu.sync_copy(rows[b], acc_sh.at[idxs[b]], add=True)
            return carry

        lax.fori_loop(0, outer, body, 0)
        plsc.subcore_barrier()
        pltpu.sync_copy(acc_sh.at[pl.ds(s * _RPT, _RPT)],
                        out_hbm.at[c, pl.ds(s * _RPT, _RPT)])

        @pl.when(s == _NS - 1)
        def _():
            pltpu.sync_copy(acc_sh.at[pl.ds(_NS * _RPT, _RTAIL)],
                            out_hbm.at[c, pl.ds(_NS * _RPT, _RTAIL)])

    return k(m, dst, zrows)


def kernel(node_feat, edge_feat, edge_index, mlp_node, mlp_edge,
           gnn1_mlp, gnn2_mlp, gnn2_mlp_edge, graph_W, graph_b):
    w1n, b1n, w2n, b2n = mlp_node
    w1e, b1e, w2e, b2e = mlp_edge
    w11, b11, w21, b21 = gnn1_mlp
    w12, b12, w22, b22 = gnn2_mlp
    we1, be1, we2, be2 = gnn2_mlp_edge

    r = lambda v: v.reshape(1, -1)
    wa1, wb1 = w11[:H], w11[H:]
    wa2, wb2 = w12[:H], w12[H:]

    dst = edge_index[1].astype(jnp.int32)

    a1 = _node_pre(node_feat, w1n, r(b1n), w2n, r(b2n), wa1)
    g1 = _sc_gather(a1, dst)

    ef, e2, m1 = _edge_fused(edge_feat.T, g1, w1e, r(b1e), w2e, r(b2e),
                             we1, r(be1), we2, r(be2),
                             wb1, r(b11), w21, r(b21), wb2, r(b12))
    p = _sc_scatter(m1, dst)

    a2 = _node_pre2(p, wa2)
    g2 = _sc_gather(a2, dst)
    m2 = _msg2(g2, e2, w22, r(b22))
    q = _sc_scatter(m2, dst)

    g = _graph_linear(q.reshape(_NC, N * H), graph_W, r(graph_b))
    return (g.reshape(H), ef)
